# Initial kernel scaffold; baseline (speedup 1.0000x reference)
import functools

import jax
import jax.numpy as jnp
from jax import lax
from jax.experimental import pallas as pl
from jax.experimental.pallas import tpu as pltpu
from jax.experimental.pallas import tpu_sc as plsc

BATCH = 16384
EMB_DIM = 128
NW = 32
B_PER_W = BATCH // NW
CHUNK = 128
N_CHUNKS = B_PER_W // CHUNK
VPR = EMB_DIM // 16


def _body(uidx_hbm, iidx_hbm, uid_table_hbm, iid_table_hbm, out_hbm,
          uidx_v, iidx_v, u_buf, v_buf, acc_buf, out_buf, sem_u, sem_v):
    wid = lax.axis_index("s") * 2 + lax.axis_index("c")
    base = wid * N_CHUNKS

    pltpu.sync_copy(uidx_hbm.at[pl.ds(base, N_CHUNKS)], uidx_v)
    pltpu.sync_copy(iidx_hbm.at[pl.ds(base, N_CHUNKS)], iidx_v)

    lanes = lax.iota(jnp.int32, 16)

    def chunk_body(j):
        cp_u = pltpu.make_async_copy(uid_table_hbm.at[uidx_v.at[j]], u_buf, sem_u)
        cp_v = pltpu.make_async_copy(iid_table_hbm.at[iidx_v.at[j]], v_buf, sem_v)
        cp_u.start()
        cp_v.start()
        cp_u.wait()
        cp_v.wait()

        def group_body(g):
            r0 = g * 16
            for i in range(16):
                row = r0 + i
                acc = u_buf[row, pl.ds(0, 16)] * v_buf[row, pl.ds(0, 16)]
                for k in range(1, VPR):
                    acc += u_buf[row, pl.ds(16 * k, 16)] * v_buf[row, pl.ds(16 * k, 16)]
                acc_buf[i, :] = acc
            out16 = plsc.load_gather(acc_buf, [lanes, jnp.zeros((16,), jnp.int32)])
            for k in range(1, 16):
                out16 += plsc.load_gather(
                    acc_buf, [lanes, jnp.full((16,), k, jnp.int32)])
            out_buf[pl.ds(r0, 16)] = out16

        pl.loop(0, CHUNK // 16)(group_body)
        pltpu.sync_copy(out_buf, out_hbm.at[pl.ds(wid * B_PER_W + j * CHUNK, CHUNK)])

    pl.loop(0, N_CHUNKS)(chunk_body)


@jax.jit
def kernel(x, uid_table, iid_table):
    uidx = x[:, 0].reshape(NW * N_CHUNKS, CHUNK).astype(jnp.int32)
    iidx = x[:, 1].reshape(NW * N_CHUNKS, CHUNK).astype(jnp.int32)

    mesh = plsc.VectorSubcoreMesh(core_axis_name="c", subcore_axis_name="s")
    run = pl.kernel(
        _body,
        out_type=jax.ShapeDtypeStruct((BATCH,), jnp.float32),
        mesh=mesh,
        scratch_types=[
            pltpu.VMEM((N_CHUNKS, CHUNK), jnp.int32),
            pltpu.VMEM((N_CHUNKS, CHUNK), jnp.int32),
            pltpu.VMEM((CHUNK, EMB_DIM), jnp.float32),
            pltpu.VMEM((CHUNK, EMB_DIM), jnp.float32),
            pltpu.VMEM((16, 16), jnp.float32),
            pltpu.VMEM((CHUNK,), jnp.float32),
            pltpu.SemaphoreType.DMA,
            pltpu.SemaphoreType.DMA,
        ],
    )
    return run(uidx, iidx, uid_table, iid_table)


# speedup vs baseline: 1.1884x; 1.1884x over previous
import functools

import jax
import jax.numpy as jnp
from jax import lax
from jax.experimental import pallas as pl
from jax.experimental.pallas import tpu as pltpu
from jax.experimental.pallas import tpu_sc as plsc

BATCH = 16384
EMB_DIM = 128
NW = 32
B_PER_W = BATCH // NW
CHUNK = 128
N_CHUNKS = B_PER_W // CHUNK
VPR = EMB_DIM // 16


def _body(uidx_hbm, iidx_hbm, uid_table_hbm, iid_table_hbm, out_hbm,
          uidx_v, iidx_v, u_buf, v_buf, out_buf, sem_u, sem_v):
    wid = lax.axis_index("s") * 2 + lax.axis_index("c")
    base = wid * N_CHUNKS

    pltpu.sync_copy(uidx_hbm.at[pl.ds(base, N_CHUNKS)], uidx_v)
    pltpu.sync_copy(iidx_hbm.at[pl.ds(base, N_CHUNKS)], iidx_v)

    lanes = lax.iota(jnp.int32, 16)

    def chunk_body(j):
        cp_u = pltpu.make_async_copy(uid_table_hbm.at[uidx_v.at[j]], u_buf, sem_u)
        cp_v = pltpu.make_async_copy(iid_table_hbm.at[iidx_v.at[j]], v_buf, sem_v)
        cp_u.start()
        cp_v.start()
        cp_u.wait()
        cp_v.wait()

        last_lane = lanes == 15

        def group_body(g):
            r0 = g * 16
            for i in range(16):
                row = r0 + i
                acc = u_buf[row, pl.ds(0, 16)] * v_buf[row, pl.ds(0, 16)]
                for k in range(1, VPR):
                    acc += u_buf[row, pl.ds(16 * k, 16)] * v_buf[row, pl.ds(16 * k, 16)]
                cum = plsc.cumsum(acc)
                plsc.store_compressed(out_buf.at[pl.ds(row, 16)], cum,
                                      mask=last_lane)

        pl.loop(0, CHUNK // 16)(group_body)
        pltpu.sync_copy(out_buf.at[pl.ds(0, CHUNK)],
                        out_hbm.at[pl.ds(wid * B_PER_W + j * CHUNK, CHUNK)])

    pl.loop(0, N_CHUNKS)(chunk_body)


@jax.jit
def kernel(x, uid_table, iid_table):
    uidx = x[:, 0].reshape(NW * N_CHUNKS, CHUNK).astype(jnp.int32)
    iidx = x[:, 1].reshape(NW * N_CHUNKS, CHUNK).astype(jnp.int32)

    mesh = plsc.VectorSubcoreMesh(core_axis_name="c", subcore_axis_name="s")
    run = pl.kernel(
        _body,
        out_type=jax.ShapeDtypeStruct((BATCH,), jnp.float32),
        mesh=mesh,
        compiler_params=pltpu.CompilerParams(needs_layout_passes=False),
        scratch_types=[
            pltpu.VMEM((N_CHUNKS, CHUNK), jnp.int32),
            pltpu.VMEM((N_CHUNKS, CHUNK), jnp.int32),
            pltpu.VMEM((CHUNK, EMB_DIM), jnp.float32),
            pltpu.VMEM((CHUNK, EMB_DIM), jnp.float32),
            pltpu.VMEM((CHUNK + 16,), jnp.float32),
            pltpu.SemaphoreType.DMA,
            pltpu.SemaphoreType.DMA,
        ],
    )
    return run(uidx, iidx, uid_table, iid_table)


# speedup vs baseline: 1.2663x; 1.0656x over previous
import jax
import jax.numpy as jnp
from jax import lax
from jax.experimental import pallas as pl
from jax.experimental.pallas import tpu as pltpu
from jax.experimental.pallas import tpu_sc as plsc

BATCH = 16384
EMB_DIM = 128
NW = 32
B_PER_W = BATCH // NW
CHUNK = 128
N_CHUNKS = B_PER_W // CHUNK
VPR = EMB_DIM // 16


def _body(uidx_hbm, iidx_hbm, uid_table_hbm, iid_table_hbm, out_hbm,
          uidx_v, iidx_v, u0, u1, v0, v1, out_buf,
          su0, su1, sv0, sv1):
    wid = lax.axis_index("s") * 2 + lax.axis_index("c")
    base = wid * N_CHUNKS

    pltpu.sync_copy(uidx_hbm.at[pl.ds(base, N_CHUNKS)], uidx_v)
    pltpu.sync_copy(iidx_hbm.at[pl.ds(base, N_CHUNKS)], iidx_v)

    u_slots, v_slots = (u0, u1), (v0, v1)
    su, sv = (su0, su1), (sv0, sv1)

    def start(j):
        s = j % 2
        cu = pltpu.make_async_copy(uid_table_hbm.at[uidx_v.at[j]], u_slots[s], su[s])
        cv = pltpu.make_async_copy(iid_table_hbm.at[iidx_v.at[j]], v_slots[s], sv[s])
        cu.start()
        cv.start()
        return cu, cv

    lanes = lax.iota(jnp.int32, 16)
    last_lane = lanes == 15

    pend = start(0)
    for j in range(N_CHUNKS):
        s = j % 2
        cu, cv = pend
        if j + 1 < N_CHUNKS:
            pend = start(j + 1)
        cu.wait()
        cv.wait()
        u_buf, v_buf = u_slots[s], v_slots[s]

        def group_body(g, u_buf=u_buf, v_buf=v_buf):
            r0 = g * 16
            for i in range(16):
                row = r0 + i
                acc = u_buf[row, pl.ds(0, 16)] * v_buf[row, pl.ds(0, 16)]
                for k in range(1, VPR):
                    acc += u_buf[row, pl.ds(16 * k, 16)] * v_buf[row, pl.ds(16 * k, 16)]
                cum = plsc.cumsum(acc)
                plsc.store_compressed(out_buf.at[pl.ds(row, 16)], cum,
                                      mask=last_lane)

        pl.loop(0, CHUNK // 16)(group_body)
        pltpu.sync_copy(out_buf.at[pl.ds(0, CHUNK)],
                        out_hbm.at[pl.ds(wid * B_PER_W + j * CHUNK, CHUNK)])


@jax.jit
def kernel(x, uid_table, iid_table):
    uidx = x[:, 0].reshape(NW * N_CHUNKS, CHUNK).astype(jnp.int32)
    iidx = x[:, 1].reshape(NW * N_CHUNKS, CHUNK).astype(jnp.int32)

    mesh = plsc.VectorSubcoreMesh(core_axis_name="c", subcore_axis_name="s")
    run = pl.kernel(
        _body,
        out_type=jax.ShapeDtypeStruct((BATCH,), jnp.float32),
        mesh=mesh,
        compiler_params=pltpu.CompilerParams(needs_layout_passes=False),
        scratch_types=[
            pltpu.VMEM((N_CHUNKS, CHUNK), jnp.int32),
            pltpu.VMEM((N_CHUNKS, CHUNK), jnp.int32),
            pltpu.VMEM((CHUNK, EMB_DIM), jnp.float32),
            pltpu.VMEM((CHUNK, EMB_DIM), jnp.float32),
            pltpu.VMEM((CHUNK, EMB_DIM), jnp.float32),
            pltpu.VMEM((CHUNK, EMB_DIM), jnp.float32),
            pltpu.VMEM((CHUNK + 16,), jnp.float32),
            pltpu.SemaphoreType.DMA,
            pltpu.SemaphoreType.DMA,
            pltpu.SemaphoreType.DMA,
            pltpu.SemaphoreType.DMA,
        ],
    )
    return run(uidx, iidx, uid_table, iid_table)


# speedup vs baseline: 1.3156x; 1.0389x over previous
import jax
import jax.numpy as jnp
from jax import lax
from jax.experimental import pallas as pl
from jax.experimental.pallas import tpu as pltpu
from jax.experimental.pallas import tpu_sc as plsc

BATCH = 16384
EMB_DIM = 128
NW = 32
B_PER_W = BATCH // NW
CHUNK = 128
N_CHUNKS = B_PER_W // CHUNK
VPR = EMB_DIM // 16
ROW_UNROLL = 8


def _body(uidx_hbm, iidx_hbm, uid_table_hbm, iid_table_hbm, out_hbm,
          uidx_v, iidx_v, u0, u1, v0, v1, out_buf,
          su0, su1, sv0, sv1):
    wid = lax.axis_index("s") * 2 + lax.axis_index("c")
    base = wid * N_CHUNKS

    pltpu.sync_copy(uidx_hbm.at[pl.ds(base, N_CHUNKS)], uidx_v)
    pltpu.sync_copy(iidx_hbm.at[pl.ds(base, N_CHUNKS)], iidx_v)

    u_slots, v_slots = (u0, u1), (v0, v1)
    su, sv = (su0, su1), (sv0, sv1)

    def start(j, s):
        pltpu.make_async_copy(
            uid_table_hbm.at[uidx_v.at[j]], u_slots[s], su[s]).start()
        pltpu.make_async_copy(
            iid_table_hbm.at[iidx_v.at[j]], v_slots[s], sv[s]).start()

    def wait(s):
        pltpu.make_async_copy(
            uid_table_hbm.at[uidx_v.at[0]], u_slots[s], su[s]).wait()
        pltpu.make_async_copy(
            iid_table_hbm.at[iidx_v.at[0]], v_slots[s], sv[s]).wait()

    lanes = lax.iota(jnp.int32, 16)
    last_lane = lanes == 15

    def compute(c, s):
        u_buf, v_buf = u_slots[s], v_slots[s]

        def group_body(g):
            r0 = g * ROW_UNROLL
            for i in range(ROW_UNROLL):
                row = r0 + i
                acc = u_buf[row, pl.ds(0, 16)] * v_buf[row, pl.ds(0, 16)]
                for k in range(1, VPR):
                    acc += u_buf[row, pl.ds(16 * k, 16)] * v_buf[row, pl.ds(16 * k, 16)]
                cum = plsc.cumsum(acc)
                plsc.store_compressed(out_buf.at[pl.ds(row, 16)], cum,
                                      mask=last_lane)

        pl.loop(0, CHUNK // ROW_UNROLL)(group_body)
        pltpu.sync_copy(out_buf.at[pl.ds(0, CHUNK)],
                        out_hbm.at[pl.ds(wid * B_PER_W + c * CHUNK, CHUNK)])

    start(0, 0)

    def pair_body(p):
        c0 = 2 * p
        start(c0 + 1, 1)
        wait(0)
        compute(c0, 0)

        @pl.when(c0 + 2 < N_CHUNKS)
        def _():
            start(c0 + 2, 0)

        wait(1)
        compute(c0 + 1, 1)

    pl.loop(0, N_CHUNKS // 2)(pair_body)


@jax.jit
def kernel(x, uid_table, iid_table):
    uidx = x[:, 0].reshape(NW * N_CHUNKS, CHUNK).astype(jnp.int32)
    iidx = x[:, 1].reshape(NW * N_CHUNKS, CHUNK).astype(jnp.int32)

    mesh = plsc.VectorSubcoreMesh(core_axis_name="c", subcore_axis_name="s")
    run = pl.kernel(
        _body,
        out_type=jax.ShapeDtypeStruct((BATCH,), jnp.float32),
        mesh=mesh,
        compiler_params=pltpu.CompilerParams(needs_layout_passes=False),
        scratch_types=[
            pltpu.VMEM((N_CHUNKS, CHUNK), jnp.int32),
            pltpu.VMEM((N_CHUNKS, CHUNK), jnp.int32),
            pltpu.VMEM((CHUNK, EMB_DIM), jnp.float32),
            pltpu.VMEM((CHUNK, EMB_DIM), jnp.float32),
            pltpu.VMEM((CHUNK, EMB_DIM), jnp.float32),
            pltpu.VMEM((CHUNK, EMB_DIM), jnp.float32),
            pltpu.VMEM((CHUNK + 16,), jnp.float32),
            pltpu.SemaphoreType.DMA,
            pltpu.SemaphoreType.DMA,
            pltpu.SemaphoreType.DMA,
            pltpu.SemaphoreType.DMA,
        ],
    )
    return run(uidx, iidx, uid_table, iid_table)


# speedup vs baseline: 1.4040x; 1.0672x over previous
import jax
import jax.numpy as jnp
from jax import lax
from jax.experimental import pallas as pl
from jax.experimental.pallas import tpu as pltpu
from jax.experimental.pallas import tpu_sc as plsc

BATCH = 16384
EMB_DIM = 128
NW = 32
B_PER_W = BATCH // NW
CHUNK = 128
N_CHUNKS = B_PER_W // CHUNK
VPR = EMB_DIM // 16
ROW_UNROLL = 8


def _body(uidx_hbm, iidx_hbm, uid_table_hbm, iid_table_hbm, out_hbm,
          uidx_v, iidx_v, u_bufs, v_bufs, out_buf, sem_u, sem_v):
    wid = lax.axis_index("s") * 2 + lax.axis_index("c")
    base = wid * N_CHUNKS

    pltpu.sync_copy(uidx_hbm.at[pl.ds(base, N_CHUNKS)], uidx_v)
    pltpu.sync_copy(iidx_hbm.at[pl.ds(base, N_CHUNKS)], iidx_v)

    def start(j, s):
        pltpu.make_async_copy(
            uid_table_hbm.at[uidx_v.at[j]], u_bufs.at[s], sem_u).start()
        pltpu.make_async_copy(
            iid_table_hbm.at[iidx_v.at[j]], v_bufs.at[s], sem_v).start()

    def wait(s):
        pltpu.make_async_copy(
            uid_table_hbm.at[uidx_v.at[0]], u_bufs.at[s], sem_u).wait()
        pltpu.make_async_copy(
            iid_table_hbm.at[iidx_v.at[0]], v_bufs.at[s], sem_v).wait()

    lanes = lax.iota(jnp.int32, 16)
    last_lane = lanes == 15

    start(0, 0)

    def chunk_body(j):
        s = lax.rem(j, 2)
        wait(s)

        @pl.when(j + 1 < N_CHUNKS)
        def _():
            start(j + 1, 1 - s)

        def group_body(g):
            r0 = g * ROW_UNROLL
            for i in range(ROW_UNROLL):
                row = r0 + i
                acc = u_bufs[s, row, pl.ds(0, 16)] * v_bufs[s, row, pl.ds(0, 16)]
                for k in range(1, VPR):
                    acc += (u_bufs[s, row, pl.ds(16 * k, 16)]
                            * v_bufs[s, row, pl.ds(16 * k, 16)])
                cum = plsc.cumsum(acc)
                plsc.store_compressed(out_buf.at[pl.ds(row, 16)], cum,
                                      mask=last_lane)

        pl.loop(0, CHUNK // ROW_UNROLL)(group_body)
        pltpu.sync_copy(out_buf.at[pl.ds(0, CHUNK)],
                        out_hbm.at[pl.ds(wid * B_PER_W + j * CHUNK, CHUNK)])

    pl.loop(0, N_CHUNKS)(chunk_body)


@jax.jit
def kernel(x, uid_table, iid_table):
    uidx = x[:, 0].reshape(NW * N_CHUNKS, CHUNK).astype(jnp.int32)
    iidx = x[:, 1].reshape(NW * N_CHUNKS, CHUNK).astype(jnp.int32)

    mesh = plsc.VectorSubcoreMesh(core_axis_name="c", subcore_axis_name="s")
    run = pl.kernel(
        _body,
        out_type=jax.ShapeDtypeStruct((BATCH,), jnp.float32),
        mesh=mesh,
        compiler_params=pltpu.CompilerParams(needs_layout_passes=False),
        scratch_types=[
            pltpu.VMEM((N_CHUNKS, CHUNK), jnp.int32),
            pltpu.VMEM((N_CHUNKS, CHUNK), jnp.int32),
            pltpu.VMEM((2, CHUNK, EMB_DIM), jnp.float32),
            pltpu.VMEM((2, CHUNK, EMB_DIM), jnp.float32),
            pltpu.VMEM((CHUNK + 16,), jnp.float32),
            pltpu.SemaphoreType.DMA,
            pltpu.SemaphoreType.DMA,
        ],
    )
    return run(uidx, iidx, uid_table, iid_table)


# speedup vs baseline: 1.4268x; 1.0162x over previous
import jax
import jax.numpy as jnp
from jax import lax
from jax.experimental import pallas as pl
from jax.experimental.pallas import tpu as pltpu
from jax.experimental.pallas import tpu_sc as plsc

BATCH = 16384
EMB_DIM = 128
NW = 32
B_PER_W = BATCH // NW
CHUNK = 128
N_CHUNKS = B_PER_W // CHUNK
VPR = EMB_DIM // 16
ROW_UNROLL = 4


def _body(uidx_hbm, iidx_hbm, uid_table_hbm, iid_table_hbm, out_hbm,
          uidx_v, iidx_v, u_bufs, v_bufs, out_buf, sem_u, sem_v):
    wid = lax.axis_index("s") * 2 + lax.axis_index("c")
    base = wid * N_CHUNKS

    pltpu.sync_copy(uidx_hbm.at[pl.ds(base, N_CHUNKS)], uidx_v)
    pltpu.sync_copy(iidx_hbm.at[pl.ds(base, N_CHUNKS)], iidx_v)

    def start(j, s):
        pltpu.make_async_copy(
            uid_table_hbm.at[uidx_v.at[j]], u_bufs.at[s], sem_u).start()
        pltpu.make_async_copy(
            iid_table_hbm.at[iidx_v.at[j]], v_bufs.at[s], sem_v).start()

    def wait(s):
        pltpu.make_async_copy(
            uid_table_hbm.at[uidx_v.at[0]], u_bufs.at[s], sem_u).wait()
        pltpu.make_async_copy(
            iid_table_hbm.at[iidx_v.at[0]], v_bufs.at[s], sem_v).wait()

    lanes = lax.iota(jnp.int32, 16)
    last_lane = lanes == 15

    start(0, 0)

    def chunk_body(j):
        s = lax.rem(j, 2)
        wait(s)

        @pl.when(j + 1 < N_CHUNKS)
        def _():
            start(j + 1, 1 - s)

        def group_body(g):
            r0 = g * ROW_UNROLL
            for i in range(ROW_UNROLL):
                row = r0 + i
                acc = u_bufs[s, row, pl.ds(0, 16)] * v_bufs[s, row, pl.ds(0, 16)]
                for k in range(1, VPR):
                    acc += (u_bufs[s, row, pl.ds(16 * k, 16)]
                            * v_bufs[s, row, pl.ds(16 * k, 16)])
                cum = plsc.cumsum(acc)
                plsc.store_compressed(out_buf.at[pl.ds(row, 16)], cum,
                                      mask=last_lane)

        pl.loop(0, CHUNK // ROW_UNROLL)(group_body)
        pltpu.sync_copy(out_buf.at[pl.ds(0, CHUNK)],
                        out_hbm.at[pl.ds(wid * B_PER_W + j * CHUNK, CHUNK)])

    pl.loop(0, N_CHUNKS)(chunk_body)


@jax.jit
def kernel(x, uid_table, iid_table):
    uidx = x[:, 0].reshape(NW * N_CHUNKS, CHUNK).astype(jnp.int32)
    iidx = x[:, 1].reshape(NW * N_CHUNKS, CHUNK).astype(jnp.int32)

    mesh = plsc.VectorSubcoreMesh(core_axis_name="c", subcore_axis_name="s")
    run = pl.kernel(
        _body,
        out_type=jax.ShapeDtypeStruct((BATCH,), jnp.float32),
        mesh=mesh,
        compiler_params=pltpu.CompilerParams(needs_layout_passes=False),
        scratch_types=[
            pltpu.VMEM((N_CHUNKS, CHUNK), jnp.int32),
            pltpu.VMEM((N_CHUNKS, CHUNK), jnp.int32),
            pltpu.VMEM((2, CHUNK, EMB_DIM), jnp.float32),
            pltpu.VMEM((2, CHUNK, EMB_DIM), jnp.float32),
            pltpu.VMEM((CHUNK + 16,), jnp.float32),
            pltpu.SemaphoreType.DMA,
            pltpu.SemaphoreType.DMA,
        ],
    )
    return run(uidx, iidx, uid_table, iid_table)


# speedup vs baseline: 1.4433x; 1.0116x over previous
import jax
import jax.numpy as jnp
from jax import lax
from jax.experimental import pallas as pl
from jax.experimental.pallas import tpu as pltpu
from jax.experimental.pallas import tpu_sc as plsc

BATCH = 16384
EMB_DIM = 128
NW = 32
B_PER_W = BATCH // NW
CHUNK = 128
N_CHUNKS = B_PER_W // CHUNK
VPR = EMB_DIM // 16
ROW_UNROLL = 2


def _body(idx_hbm, uid_table_hbm, iid_table_hbm, out_hbm,
          idx_v, u_bufs, v_bufs, out_buf, sem_u, sem_v):
    wid = lax.axis_index("s") * 2 + lax.axis_index("c")
    base = wid * 2 * N_CHUNKS

    pltpu.sync_copy(idx_hbm.at[pl.ds(base, 2 * N_CHUNKS)], idx_v)

    def start(j, s):
        pltpu.make_async_copy(
            uid_table_hbm.at[idx_v.at[j]], u_bufs.at[s], sem_u).start()
        pltpu.make_async_copy(
            iid_table_hbm.at[idx_v.at[N_CHUNKS + j]], v_bufs.at[s], sem_v).start()

    def wait(s):
        pltpu.make_async_copy(
            uid_table_hbm.at[idx_v.at[0]], u_bufs.at[s], sem_u).wait()
        pltpu.make_async_copy(
            iid_table_hbm.at[idx_v.at[0]], v_bufs.at[s], sem_v).wait()

    lanes = lax.iota(jnp.int32, 16)
    last_lane = lanes == 15

    start(0, 0)

    def chunk_body(j):
        s = lax.rem(j, 2)
        wait(s)

        @pl.when(j + 1 < N_CHUNKS)
        def _():
            start(j + 1, 1 - s)

        def group_body(g):
            r0 = g * ROW_UNROLL
            for i in range(ROW_UNROLL):
                row = r0 + i
                acc = u_bufs[s, row, pl.ds(0, 16)] * v_bufs[s, row, pl.ds(0, 16)]
                for k in range(1, VPR):
                    acc += (u_bufs[s, row, pl.ds(16 * k, 16)]
                            * v_bufs[s, row, pl.ds(16 * k, 16)])
                cum = plsc.cumsum(acc)
                plsc.store_compressed(out_buf.at[pl.ds(row, 16)], cum,
                                      mask=last_lane)

        pl.loop(0, CHUNK // ROW_UNROLL)(group_body)
        pltpu.sync_copy(out_buf.at[pl.ds(0, CHUNK)],
                        out_hbm.at[pl.ds(wid * B_PER_W + j * CHUNK, CHUNK)])

    pl.loop(0, N_CHUNKS)(chunk_body)


@jax.jit
def kernel(x, uid_table, iid_table):
    idx = (x.astype(jnp.int32)
           .reshape(NW, N_CHUNKS, CHUNK, 2)
           .transpose(0, 3, 1, 2)
           .reshape(NW * 2 * N_CHUNKS, CHUNK))

    mesh = plsc.VectorSubcoreMesh(core_axis_name="c", subcore_axis_name="s")
    run = pl.kernel(
        _body,
        out_type=jax.ShapeDtypeStruct((BATCH,), jnp.float32),
        mesh=mesh,
        compiler_params=pltpu.CompilerParams(needs_layout_passes=False),
        scratch_types=[
            pltpu.VMEM((2 * N_CHUNKS, CHUNK), jnp.int32),
            pltpu.VMEM((2, CHUNK, EMB_DIM), jnp.float32),
            pltpu.VMEM((2, CHUNK, EMB_DIM), jnp.float32),
            pltpu.VMEM((CHUNK + 16,), jnp.float32),
            pltpu.SemaphoreType.DMA,
            pltpu.SemaphoreType.DMA,
        ],
    )
    return run(idx, uid_table, iid_table)
